# Initial kernel scaffold; baseline (speedup 1.0000x reference)
import jax
import jax.numpy as jnp
from jax.experimental import pallas as pl


def kernel(s_f, t_f, gallery, mask_init):
    raise NotImplementedError("write your pallas kernel here")


# speedup vs baseline: 33.8580x; 33.8580x over previous
import functools

import jax
import jax.numpy as jnp
from jax import lax
from jax.experimental import pallas as pl
from jax.experimental.pallas import tpu as pltpu
from jax.experimental.pallas import tpu_sc as plsc

_SIZE = 100000
_B = 128
_L = 128
_K = 5
_KEEP = 64

_CHUNK = 2048
_NCHUNK = (_SIZE + _CHUNK - 1) // _CHUNK

_INF = float("inf")
_IMAX = 2**31 - 1

_GROWS = _K * _B + 128
_NW = 32
_BPW = _GROWS // _NW


def _extract5(vals, ids):
    cols_v, cols_i = [], []
    for _ in range(_K):
        m = jnp.min(vals, axis=1, keepdims=True)
        pick = jnp.min(jnp.where(vals == m, ids, _IMAX), axis=1, keepdims=True)
        cols_v.append(m)
        cols_i.append(pick)
        vals = jnp.where(ids == pick, _INF, vals)
    pad_v = jnp.full((_B, 8 - _K), _INF, jnp.float32)
    pad_i = jnp.full((_B, 8 - _K), _IMAX, jnp.int32)
    return (jnp.concatenate(cols_v + [pad_v], axis=1),
            jnp.concatenate(cols_i + [pad_i], axis=1))


def _topk_body(t_ref, g_ref, out_ref, bestv_ref, besti_ref):
    i = pl.program_id(0)
    t = t_ref[...]
    xx = jnp.sum(t * t, axis=1, keepdims=True)
    ones_row = jnp.ones((1, _L), jnp.float32)

    @pl.when(i == 0)
    def _init():
        mm = lax.dot_general(t, t, (((1,), (1,)), ((), ())),
                             preferred_element_type=jnp.float32)
        xs_row = lax.dot_general(ones_row, t * t, (((1,), (1,)), ((), ())),
                                 preferred_element_type=jnp.float32)
        d = jnp.maximum(xx + xs_row - 2.0 * mm, 1e-12)
        ids = lax.broadcasted_iota(jnp.int32, (_B, _B), 1)
        v5, i5 = _extract5(d, ids)
        bestv_ref[...] = v5
        besti_ref[...] = i5

    g = g_ref[...]
    mm = lax.dot_general(t, g, (((1,), (1,)), ((), ())),
                         preferred_element_type=jnp.float32)
    yy_row = lax.dot_general(ones_row, g * g, (((1,), (1,)), ((), ())),
                             preferred_element_type=jnp.float32)
    d = jnp.maximum(xx + yy_row - 2.0 * mm, 1e-12)
    ids = i * _CHUNK + lax.broadcasted_iota(jnp.int32, (_B, _CHUNK), 1)
    d = jnp.where((ids >= _B) & (ids < _SIZE), d, _INF)
    cv, ci = _extract5(d, ids)
    nv, ni = _extract5(jnp.concatenate([cv, bestv_ref[...]], axis=1),
                       jnp.concatenate([ci, besti_ref[...]], axis=1))
    bestv_ref[...] = nv
    besti_ref[...] = ni

    @pl.when(i == _NCHUNK - 1)
    def _emit():
        out_ref[...] = besti_ref[...]


def _knn_top5(t_f, gallery):
    return pl.pallas_call(
        _topk_body,
        grid=(_NCHUNK,),
        in_specs=[
            pl.BlockSpec((_B, _L), lambda i: (0, 0)),
            pl.BlockSpec((_CHUNK, _L), lambda i: (i, 0)),
        ],
        out_specs=pl.BlockSpec((_B, 8), lambda i: (0, 0)),
        out_shape=jax.ShapeDtypeStruct((_B, 8), jnp.int32),
        scratch_shapes=[
            pltpu.VMEM((_B, 8), jnp.float32),
            pltpu.VMEM((_B, 8), jnp.int32),
        ],
    )(t_f, gallery)


def _gather_rows(gallery, idx):
    mesh = plsc.VectorSubcoreMesh(core_axis_name="c", subcore_axis_name="s")

    @functools.partial(
        pl.kernel,
        mesh=mesh,
        out_type=jax.ShapeDtypeStruct((_GROWS, _L), jnp.float32),
        scratch_types=[
            pltpu.VMEM((_BPW,), jnp.int32),
            pltpu.VMEM((_BPW, _L), jnp.float32),
            pltpu.SemaphoreType.DMA,
        ],
    )
    def gather_k(gallery_hbm, idx_hbm, out_hbm, idx_v, rows_v, sem):
        wid = lax.axis_index("s") * 2 + lax.axis_index("c")
        base = wid * _BPW
        pltpu.sync_copy(idx_hbm.at[pl.ds(base, _BPW)], idx_v)
        pltpu.async_copy(gallery_hbm.at[idx_v], rows_v, sem).wait()
        pltpu.sync_copy(rows_v, out_hbm.at[pl.ds(base, _BPW)])

    return gather_k(gallery, idx)


def _mask_body(sim_ref, idx_ref, s_ref, t_ref, mask_ref, out_ref):
    s = s_ref[...]
    t = t_ref[...]
    iota_col = lax.broadcasted_iota(jnp.int32, (_L, 1), 0)
    eye = (lax.broadcasted_iota(jnp.int32, (_L, _L), 0) ==
           lax.broadcasted_iota(jnp.int32, (_L, _L), 1)).astype(jnp.float32)
    idx_all = idx_ref[...]

    blocks = []
    for kk in range(_K):
        rows = sim_ref[kk * _B:(kk + 1) * _B, :]
        idxrow = idx_all[kk:kk + 1, :]
        ohT = (iota_col == idxrow).astype(jnp.float32)
        fix = lax.dot_general(ohT, t, (((0,), (0,)), ((), ())),
                              preferred_element_type=jnp.float32)
        matchrow = jnp.sum(ohT, axis=0, keepdims=True)
        matchcol = lax.dot_general(eye, matchrow, (((1,), (1,)), ((), ())),
                                   preferred_element_type=jnp.float32)
        fixed = fix + rows * (1.0 - matchcol)
        blocks.append(jnp.abs(fixed * s))
    dall = jnp.concatenate(blocks, axis=0)
    lane = lax.broadcasted_iota(jnp.int32, (_K * _B, _L), 1)

    def body(l, member):
        e = (iota_col == l).astype(jnp.float32)
        col = lax.dot_general(dall, e, (((1,), (0,)), ((), ())),
                              preferred_element_type=jnp.float32)
        lt = jnp.sum((dall < col).astype(jnp.float32), axis=1, keepdims=True)
        eqb = jnp.sum(((dall == col) & (lane < l)).astype(jnp.float32),
                      axis=1, keepdims=True)
        intop = ((lt + eqb) < _KEEP).astype(jnp.float32)
        allb = jnp.min(intop)
        lrow = lax.broadcasted_iota(jnp.int32, (1, _L), 1)
        return member * jnp.where(lrow == l, allb, 1.0)

    member = lax.fori_loop(0, _L, body, jnp.ones((1, _L), jnp.float32))
    maskrow = jnp.where(member > 0.5, 0.0, mask_ref[...])
    maskcol = lax.dot_general(eye, maskrow, (((1,), (1,)), ((), ())),
                              preferred_element_type=jnp.float32)
    out_ref[...] = jnp.broadcast_to(maskcol, (_L, _L))


def _mask_compute(sim, idx2d, s_f, t_f, mask_row):
    return pl.pallas_call(
        _mask_body,
        out_shape=jax.ShapeDtypeStruct((_L, _L), jnp.float32),
    )(sim, idx2d, s_f, t_f, mask_row)


def kernel(s_f, t_f, gallery, mask_init):
    top8 = _knn_top5(t_f, gallery)
    idx2d = top8[:, :_K].T
    idx_flat = jnp.concatenate(
        [idx2d.reshape(_K * _B), jnp.zeros((_GROWS - _K * _B,), jnp.int32)])
    sim = _gather_rows(gallery, idx_flat)
    out2d = _mask_compute(sim, idx2d, s_f, t_f, mask_init.reshape(1, _L))
    return out2d.reshape(_L, _L, 1, 1)


# speedup vs baseline: 40.2231x; 1.1880x over previous
import functools

import jax
import jax.numpy as jnp
from jax import lax
from jax.experimental import pallas as pl
from jax.experimental.pallas import tpu as pltpu
from jax.experimental.pallas import tpu_sc as plsc

_SIZE = 100000
_B = 128
_L = 128
_K = 5
_KEEP = 64

_CHUNK = 8192
_NCHUNK = (_SIZE + _CHUNK - 1) // _CHUNK
_NSEG = _CHUNK // _L

_INF = float("inf")
_BIGID = float(2**30)

_IDXROWS = 6
_GROWS = _IDXROWS * _B
_NW = 32
_BPW = _GROWS // _NW


def _extract5(vals, ids):
    cols_v, cols_i = [], []
    for _ in range(_K):
        m = jnp.min(vals, axis=1, keepdims=True)
        pick = jnp.min(jnp.where(vals == m, ids, _BIGID), axis=1, keepdims=True)
        cols_v.append(m)
        cols_i.append(pick)
        vals = jnp.where(ids == pick, _INF, vals)
    pad_v = jnp.full((_B, 8 - _K), _INF, jnp.float32)
    pad_i = jnp.full((_B, 8 - _K), _BIGID, jnp.float32)
    return (jnp.concatenate(cols_v + [pad_v], axis=1),
            jnp.concatenate(cols_i + [pad_i], axis=1))


def _topk_body(t_ref, g_ref, out_ref, bestv_ref, besti_ref,
               chunkv_ref, chunki_ref):
    i = pl.program_id(0)
    t = t_ref[...]
    xx = jnp.sum(t * t, axis=1, keepdims=True)
    ones_row = jnp.ones((1, _L), jnp.float32)

    @pl.when(i == 0)
    def _init():
        mm0 = lax.dot_general(t, t, (((1,), (1,)), ((), ())),
                              preferred_element_type=jnp.float32)
        xs_row = lax.dot_general(ones_row, t * t, (((1,), (1,)), ((), ())),
                                 preferred_element_type=jnp.float32)
        d0 = jnp.maximum(xx + xs_row - 2.0 * mm0, 1e-12)
        ids0 = lax.broadcasted_iota(jnp.int32, (_B, _B), 1).astype(jnp.float32)
        v5, i5 = _extract5(d0, ids0)
        bestv_ref[...] = v5
        besti_ref[...] = i5

    g = g_ref[...]
    mm = lax.dot_general(t, g, (((1,), (1,)), ((), ())),
                         preferred_element_type=jnp.float32)
    yy_row = lax.dot_general(ones_row, g * g, (((1,), (1,)), ((), ())),
                             preferred_element_type=jnp.float32)
    base = (i * _CHUNK).astype(jnp.float32)
    iota_l = lax.broadcasted_iota(jnp.int32, (_B, _L), 1).astype(jnp.float32)

    def seg_d(s):
        ids_s = base + (float(s * _L) + iota_l)
        ds = xx + yy_row[:, s * _L:(s + 1) * _L] - 2.0 * mm[:, s * _L:(s + 1) * _L]
        ds = jnp.maximum(ds, 1e-12)
        ds = jnp.where((ids_s >= float(_B)) & (ids_s < float(_SIZE)), ds, _INF)
        return ds, ids_s

    accv, accs = None, None
    for s in range(_NSEG):
        ds, _ = seg_d(s)
        if s == 0:
            accv = ds
            accs = jnp.zeros((_B, _L), jnp.float32)
        else:
            take = ds < accv
            accs = jnp.where(take, float(s), accs)
            accv = jnp.where(take, ds, accv)
    gids = base + accs * float(_L) + iota_l
    cv, ci = _extract5(accv, gids)
    chunkv_ref[...] = cv
    chunki_ref[...] = ci

    v5 = cv[:, 4:5]
    g5 = ci[:, 4:5]
    cnt = jnp.zeros((_B, 1), jnp.float32)
    for s in range(_NSEG):
        ds, ids_s = seg_d(s)
        lex = (ds < v5) | ((ds == v5) & (ids_s < g5))
        cnt = cnt + jnp.sum(lex.astype(jnp.float32), axis=1, keepdims=True)

    @pl.when(jnp.any(cnt != 4.0))
    def _fallback():
        d = jnp.maximum(xx + yy_row - 2.0 * mm, 1e-12)
        ids = base + lax.broadcasted_iota(jnp.int32, (_B, _CHUNK), 1).astype(
            jnp.float32)
        d = jnp.where((ids >= float(_B)) & (ids < float(_SIZE)), d, _INF)
        fv, fi = _extract5(d, ids)
        chunkv_ref[...] = fv
        chunki_ref[...] = fi

    nv, ni = _extract5(jnp.concatenate([chunkv_ref[...], bestv_ref[...]], axis=1),
                       jnp.concatenate([chunki_ref[...], besti_ref[...]], axis=1))
    bestv_ref[...] = nv
    besti_ref[...] = ni

    @pl.when(i == _NCHUNK - 1)
    def _emit():
        bi = besti_ref[...]
        eye = (lax.broadcasted_iota(jnp.int32, (_B, _B), 0) ==
               lax.broadcasted_iota(jnp.int32, (_B, _B), 1)).astype(jnp.float32)
        bi_t = lax.dot_general(bi, eye, (((0,), (0,)), ((), ())),
                               preferred_element_type=jnp.float32)
        rows = lax.broadcasted_iota(jnp.int32, (8, _B), 0)
        bi_t = jnp.where(rows < _K, bi_t, 0.0)
        out_ref[...] = bi_t[:_IDXROWS, :].astype(jnp.int32)


def _knn_top5(t_f, gallery):
    return pl.pallas_call(
        _topk_body,
        grid=(_NCHUNK,),
        in_specs=[
            pl.BlockSpec((_B, _L), lambda i: (0, 0)),
            pl.BlockSpec((_CHUNK, _L), lambda i: (i, 0)),
        ],
        out_specs=pl.BlockSpec((_IDXROWS, _B), lambda i: (0, 0)),
        out_shape=jax.ShapeDtypeStruct((_IDXROWS, _B), jnp.int32),
        scratch_shapes=[
            pltpu.VMEM((_B, 8), jnp.float32),
            pltpu.VMEM((_B, 8), jnp.float32),
            pltpu.VMEM((_B, 8), jnp.float32),
            pltpu.VMEM((_B, 8), jnp.float32),
        ],
    )(t_f, gallery)


def _gather_rows(gallery, idx):
    mesh = plsc.VectorSubcoreMesh(core_axis_name="c", subcore_axis_name="s")

    @functools.partial(
        pl.kernel,
        mesh=mesh,
        out_type=jax.ShapeDtypeStruct((_GROWS, _L), jnp.float32),
        scratch_types=[
            pltpu.VMEM((_BPW,), jnp.int32),
            pltpu.VMEM((_BPW, _L), jnp.float32),
            pltpu.SemaphoreType.DMA,
        ],
    )
    def gather_k(gallery_hbm, idx_hbm, out_hbm, idx_v, rows_v, sem):
        wid = lax.axis_index("s") * 2 + lax.axis_index("c")
        base = wid * _BPW
        pltpu.sync_copy(idx_hbm.at[pl.ds(base, _BPW)], idx_v)
        pltpu.async_copy(gallery_hbm.at[idx_v], rows_v, sem).wait()
        pltpu.sync_copy(rows_v, out_hbm.at[pl.ds(base, _BPW)])

    return gather_k(gallery, idx)


def _mask_body(sim_ref, idx_ref, s_ref, t_ref, mask_ref, out_ref):
    s = s_ref[...]
    t = t_ref[...]
    iota_col = lax.broadcasted_iota(jnp.int32, (_L, 1), 0)
    eye = (lax.broadcasted_iota(jnp.int32, (_L, _L), 0) ==
           lax.broadcasted_iota(jnp.int32, (_L, _L), 1)).astype(jnp.float32)
    idx_all = idx_ref[...]

    blocks = []
    for kk in range(_K):
        rows = sim_ref[kk * _B:(kk + 1) * _B, :]
        idxrow = idx_all[kk:kk + 1, :]
        ohT = (iota_col == idxrow).astype(jnp.float32)
        fix = lax.dot_general(ohT, t, (((0,), (0,)), ((), ())),
                              preferred_element_type=jnp.float32)
        matchrow = jnp.sum(ohT, axis=0, keepdims=True)
        matchcol = lax.dot_general(eye, matchrow, (((1,), (1,)), ((), ())),
                                   preferred_element_type=jnp.float32)
        fixed = fix + rows * (1.0 - matchcol)
        blocks.append(jnp.abs(fixed * s))
    dall = jnp.concatenate(blocks, axis=0)
    lane = lax.broadcasted_iota(jnp.int32, (_K * _B, _L), 1)

    def body(l, member):
        e = (iota_col == l).astype(jnp.float32)
        col = lax.dot_general(dall, e, (((1,), (0,)), ((), ())),
                              preferred_element_type=jnp.float32)
        lt = jnp.sum((dall < col).astype(jnp.float32), axis=1, keepdims=True)
        eqb = jnp.sum(((dall == col) & (lane < l)).astype(jnp.float32),
                      axis=1, keepdims=True)
        intop = ((lt + eqb) < _KEEP).astype(jnp.float32)
        allb = jnp.min(intop)
        lrow = lax.broadcasted_iota(jnp.int32, (1, _L), 1)
        return member * jnp.where(lrow == l, allb, 1.0)

    member = lax.fori_loop(0, _L, body, jnp.ones((1, _L), jnp.float32))
    maskrow = jnp.where(member > 0.5, 0.0, mask_ref[...])
    maskcol = lax.dot_general(eye, maskrow, (((1,), (1,)), ((), ())),
                              preferred_element_type=jnp.float32)
    out_ref[...] = jnp.broadcast_to(maskcol, (_L, _L))


def _mask_compute(sim, idx2d, s_f, t_f, mask_row):
    return pl.pallas_call(
        _mask_body,
        out_shape=jax.ShapeDtypeStruct((_L, _L), jnp.float32),
    )(sim, idx2d, s_f, t_f, mask_row)


def kernel(s_f, t_f, gallery, mask_init):
    idx2d = _knn_top5(t_f, gallery)
    sim = _gather_rows(gallery, idx2d.reshape(_GROWS))
    out2d = _mask_compute(sim, idx2d, s_f, t_f, mask_init.reshape(1, _L))
    return out2d.reshape(_L, _L, 1, 1)


# speedup vs baseline: 45.2532x; 1.1251x over previous
import functools

import jax
import jax.numpy as jnp
from jax import lax
from jax.experimental import pallas as pl
from jax.experimental.pallas import tpu as pltpu
from jax.experimental.pallas import tpu_sc as plsc

_SIZE = 100000
_B = 128
_L = 128
_K = 5
_KEEP = 64

_CHUNK = 8192
_NCHUNK = (_SIZE + _CHUNK - 1) // _CHUNK
_NSEG = _CHUNK // _L

_INF = float("inf")
_BIGID = float(2**30)

_IDXROWS = 6
_GROWS = _IDXROWS * _B
_NW = 32
_BPW = _GROWS // _NW


def _extract5(vals, ids):
    cols_v, cols_i = [], []
    for _ in range(_K):
        m = jnp.min(vals, axis=1, keepdims=True)
        pick = jnp.min(jnp.where(vals == m, ids, _BIGID), axis=1, keepdims=True)
        cols_v.append(m)
        cols_i.append(pick)
        vals = jnp.where(ids == pick, _INF, vals)
    pad_v = jnp.full((_B, 8 - _K), _INF, jnp.float32)
    pad_i = jnp.full((_B, 8 - _K), _BIGID, jnp.float32)
    return (jnp.concatenate(cols_v + [pad_v], axis=1),
            jnp.concatenate(cols_i + [pad_i], axis=1))


def _topk_body(t_ref, g_ref, out_ref, bestv_ref, besti_ref,
               chunkv_ref, chunki_ref):
    i = pl.program_id(0)
    t = t_ref[...]
    xx = jnp.sum(t * t, axis=1, keepdims=True)
    ones_row = jnp.ones((1, _L), jnp.float32)

    @pl.when(i == 0)
    def _init():
        mm0 = lax.dot_general(t, t, (((1,), (1,)), ((), ())),
                              preferred_element_type=jnp.float32)
        xs_row = lax.dot_general(ones_row, t * t, (((1,), (1,)), ((), ())),
                                 preferred_element_type=jnp.float32)
        d0 = jnp.maximum(xx + xs_row - 2.0 * mm0, 1e-12)
        ids0 = lax.broadcasted_iota(jnp.int32, (_B, _B), 1).astype(jnp.float32)
        v5, i5 = _extract5(d0, ids0)
        bestv_ref[...] = v5
        besti_ref[...] = i5

    g = g_ref[...]
    mm = lax.dot_general(t, g, (((1,), (1,)), ((), ())),
                         preferred_element_type=jnp.float32)
    yy_row = lax.dot_general(ones_row, g * g, (((1,), (1,)), ((), ())),
                             preferred_element_type=jnp.float32)
    base = (i * _CHUNK).astype(jnp.float32)
    iota_l = lax.broadcasted_iota(jnp.int32, (_B, _L), 1).astype(jnp.float32)

    def seg_d(s):
        ids_s = base + (float(s * _L) + iota_l)
        ds = xx + yy_row[:, s * _L:(s + 1) * _L] - 2.0 * mm[:, s * _L:(s + 1) * _L]
        ds = jnp.maximum(ds, 1e-12)
        ds = jnp.where((ids_s >= float(_B)) & (ids_s < float(_SIZE)), ds, _INF)
        return ds, ids_s

    accv, accs = None, None
    for s in range(_NSEG):
        ds, _ = seg_d(s)
        if s == 0:
            accv = ds
            accs = jnp.zeros((_B, _L), jnp.float32)
        else:
            take = ds < accv
            accs = jnp.where(take, float(s), accs)
            accv = jnp.where(take, ds, accv)
    gids = base + accs * float(_L) + iota_l
    cv, ci = _extract5(accv, gids)
    chunkv_ref[...] = cv
    chunki_ref[...] = ci

    v5 = cv[:, 4:5]
    g5 = ci[:, 4:5]
    cnt = jnp.zeros((_B, 1), jnp.float32)
    for s in range(_NSEG):
        ds, ids_s = seg_d(s)
        lex = (ds < v5) | ((ds == v5) & (ids_s < g5))
        cnt = cnt + jnp.sum(lex.astype(jnp.float32), axis=1, keepdims=True)

    @pl.when(jnp.any(cnt != 4.0))
    def _fallback():
        d = jnp.maximum(xx + yy_row - 2.0 * mm, 1e-12)
        ids = base + lax.broadcasted_iota(jnp.int32, (_B, _CHUNK), 1).astype(
            jnp.float32)
        d = jnp.where((ids >= float(_B)) & (ids < float(_SIZE)), d, _INF)
        fv, fi = _extract5(d, ids)
        chunkv_ref[...] = fv
        chunki_ref[...] = fi

    nv, ni = _extract5(jnp.concatenate([chunkv_ref[...], bestv_ref[...]], axis=1),
                       jnp.concatenate([chunki_ref[...], besti_ref[...]], axis=1))
    bestv_ref[...] = nv
    besti_ref[...] = ni

    @pl.when(i == _NCHUNK - 1)
    def _emit():
        bi = besti_ref[...]
        eye = (lax.broadcasted_iota(jnp.int32, (_B, _B), 0) ==
               lax.broadcasted_iota(jnp.int32, (_B, _B), 1)).astype(jnp.float32)
        bi_t = lax.dot_general(bi, eye, (((0,), (0,)), ((), ())),
                               preferred_element_type=jnp.float32)
        rows = lax.broadcasted_iota(jnp.int32, (8, _B), 0)
        bi_t = jnp.where(rows < _K, bi_t, 0.0)
        out_ref[...] = bi_t[:_IDXROWS, :].astype(jnp.int32)


def _knn_top5(t_f, gallery):
    return pl.pallas_call(
        _topk_body,
        grid=(_NCHUNK,),
        in_specs=[
            pl.BlockSpec((_B, _L), lambda i: (0, 0)),
            pl.BlockSpec((_CHUNK, _L), lambda i: (i, 0)),
        ],
        out_specs=pl.BlockSpec((_IDXROWS, _B), lambda i: (0, 0)),
        out_shape=jax.ShapeDtypeStruct((_IDXROWS, _B), jnp.int32),
        scratch_shapes=[
            pltpu.VMEM((_B, 8), jnp.float32),
            pltpu.VMEM((_B, 8), jnp.float32),
            pltpu.VMEM((_B, 8), jnp.float32),
            pltpu.VMEM((_B, 8), jnp.float32),
        ],
    )(t_f, gallery)


def _gather_rows(gallery, idx):
    mesh = plsc.VectorSubcoreMesh(core_axis_name="c", subcore_axis_name="s")

    @functools.partial(
        pl.kernel,
        mesh=mesh,
        out_type=jax.ShapeDtypeStruct((_GROWS, _L), jnp.float32),
        scratch_types=[
            pltpu.VMEM((_BPW,), jnp.int32),
            pltpu.VMEM((_BPW, _L), jnp.float32),
            pltpu.SemaphoreType.DMA,
        ],
    )
    def gather_k(gallery_hbm, idx_hbm, out_hbm, idx_v, rows_v, sem):
        wid = lax.axis_index("s") * 2 + lax.axis_index("c")
        base = wid * _BPW
        pltpu.sync_copy(idx_hbm.at[pl.ds(base, _BPW)], idx_v)
        pltpu.async_copy(gallery_hbm.at[idx_v], rows_v, sem).wait()
        pltpu.sync_copy(rows_v, out_hbm.at[pl.ds(base, _BPW)])

    return gather_k(gallery, idx)


def _mask_body(sim_ref, idx_ref, s_ref, t_ref, mask_ref, out_ref):
    s = s_ref[...]
    t = t_ref[...]
    iota_col = lax.broadcasted_iota(jnp.int32, (_L, 1), 0)
    eye = (lax.broadcasted_iota(jnp.int32, (_L, _L), 0) ==
           lax.broadcasted_iota(jnp.int32, (_L, _L), 1)).astype(jnp.float32)
    idx_all = idx_ref[...]

    blocks = []
    for kk in range(_K):
        rows = sim_ref[kk * _B:(kk + 1) * _B, :]
        idxrow = idx_all[kk:kk + 1, :]
        ohT = (iota_col == idxrow).astype(jnp.float32)
        fix = lax.dot_general(ohT, t, (((0,), (0,)), ((), ())),
                              preferred_element_type=jnp.float32)
        matchrow = jnp.sum(ohT, axis=0, keepdims=True)
        matchcol = lax.dot_general(eye, matchrow, (((1,), (1,)), ((), ())),
                                   preferred_element_type=jnp.float32)
        fixed = fix + rows * (1.0 - matchcol)
        blocks.append(jnp.abs(fixed * s))
    dall = jnp.concatenate(blocks, axis=0)
    lane = lax.broadcasted_iota(jnp.int32, (_K * _B, _L), 1)
    lrow = lax.broadcasted_iota(jnp.int32, (1, _L), 1)
    iota8 = lax.broadcasted_iota(jnp.int32, (1, 8), 1)

    def body(tt, member):
        e8 = (iota_col == tt * 8 + iota8).astype(jnp.float32)
        cols = lax.dot_general(dall, e8, (((1,), (0,)), ((), ())),
                               preferred_element_type=jnp.float32)
        for c in range(8):
            l = tt * 8 + c
            col = cols[:, c:c + 1]
            lt = jnp.sum((dall < col).astype(jnp.float32), axis=1,
                         keepdims=True)
            eqb = jnp.sum(((dall == col) & (lane < l)).astype(jnp.float32),
                          axis=1, keepdims=True)
            intop = ((lt + eqb) < _KEEP).astype(jnp.float32)
            allb = jnp.min(intop)
            member = member * jnp.where(lrow == l, allb, 1.0)
        return member

    member = lax.fori_loop(0, _L // 8, body, jnp.ones((1, _L), jnp.float32))
    maskrow = jnp.where(member > 0.5, 0.0, mask_ref[...])
    maskcol = lax.dot_general(eye, maskrow, (((1,), (1,)), ((), ())),
                              preferred_element_type=jnp.float32)
    out_ref[...] = jnp.broadcast_to(maskcol, (_L, _L))


def _mask_compute(sim, idx2d, s_f, t_f, mask_row):
    return pl.pallas_call(
        _mask_body,
        out_shape=jax.ShapeDtypeStruct((_L, _L), jnp.float32),
    )(sim, idx2d, s_f, t_f, mask_row)


def kernel(s_f, t_f, gallery, mask_init):
    idx2d = _knn_top5(t_f, gallery)
    sim = _gather_rows(gallery, idx2d.reshape(_GROWS))
    out2d = _mask_compute(sim, idx2d, s_f, t_f, mask_init.reshape(1, _L))
    return out2d.reshape(_L, _L, 1, 1)


# speedup vs baseline: 55.3514x; 1.2231x over previous
import functools

import jax
import jax.numpy as jnp
from jax import lax
from jax.experimental import pallas as pl
from jax.experimental.pallas import tpu as pltpu
from jax.experimental.pallas import tpu_sc as plsc

_SIZE = 100000
_B = 128
_L = 128
_K = 5
_KEEP = 64

_CHUNK = 8192
_NCHUNK = (_SIZE + _CHUNK - 1) // _CHUNK
_NSEG = _CHUNK // _L

_INF = float("inf")
_BIGID = float(2**30)

_IDXROWS = 6
_GROWS = _IDXROWS * _B
_NW = 32
_BPW = _GROWS // _NW


def _extract5(vals, ids):
    cols_v, cols_i = [], []
    for _ in range(_K):
        m = jnp.min(vals, axis=1, keepdims=True)
        pick = jnp.min(jnp.where(vals == m, ids, _BIGID), axis=1, keepdims=True)
        cols_v.append(m)
        cols_i.append(pick)
        vals = jnp.where(ids == pick, _INF, vals)
    pad_v = jnp.full((_B, 8 - _K), _INF, jnp.float32)
    pad_i = jnp.full((_B, 8 - _K), _BIGID, jnp.float32)
    return (jnp.concatenate(cols_v + [pad_v], axis=1),
            jnp.concatenate(cols_i + [pad_i], axis=1))


_SLAB = 8
_NSLAB = _B // _SLAB


def _topk_body(t_ref, g_ref, out_ref, stv_ref, sts_ref, mm_ref, xx_ref):
    i = pl.program_id(0)
    t = t_ref[...]
    xx = jnp.sum(t * t, axis=1, keepdims=True)
    ones_row = jnp.ones((1, _L), jnp.float32)

    @pl.when(i == 0)
    def _init():
        mm0 = lax.dot_general(t, t, (((1,), (1,)), ((), ())),
                              preferred_element_type=jnp.float32)
        xs_row = lax.dot_general(ones_row, t * t, (((1,), (1,)), ((), ())),
                                 preferred_element_type=jnp.float32)
        d0 = jnp.maximum(xx + xs_row - 2.0 * mm0, 1e-12)
        stv_ref[:, 0:_L] = d0
        stv_ref[:, _L:] = jnp.full((_B, 4 * _L), _INF, jnp.float32)
        sts_ref[...] = jnp.zeros((_B, _K * _L), jnp.float32)

    g = g_ref[...]
    mm_ref[...] = lax.dot_general(t, g, (((1,), (1,)), ((), ())),
                                  preferred_element_type=jnp.float32)
    yy_row = lax.dot_general(ones_row, g * g, (((1,), (1,)), ((), ())),
                             preferred_element_type=jnp.float32)
    xx_ref[...] = xx
    base = (i * _CHUNK).astype(jnp.float32)
    segbase = (i * _NSEG).astype(jnp.float32)
    iota8 = lax.broadcasted_iota(jnp.int32, (_SLAB, _L), 1).astype(jnp.float32)

    def slab_body(sl, carry):
        rows = pl.ds(sl * _SLAB, _SLAB)
        xx8 = xx_ref[rows, :]
        v = [stv_ref[rows, k * _L:(k + 1) * _L] for k in range(_K)]
        sg = [sts_ref[rows, k * _L:(k + 1) * _L] for k in range(_K)]
        for s in range(_NSEG):
            ids_s = base + (float(s * _L) + iota8)
            mmsl = mm_ref[rows, s * _L:(s + 1) * _L]
            ds = xx8 + yy_row[:, s * _L:(s + 1) * _L] - 2.0 * mmsl
            ds = jnp.maximum(ds, 1e-12)
            ds = jnp.where((ids_s >= float(_B)) & (ids_s < float(_SIZE)),
                           ds, _INF)
            fs = segbase + float(s)
            c = [ds < v[k] for k in range(_K)]
            for k in range(_K - 1, 0, -1):
                v[k] = jnp.where(c[k], jnp.where(c[k - 1], v[k - 1], ds), v[k])
                sg[k] = jnp.where(c[k], jnp.where(c[k - 1], sg[k - 1], fs),
                                  sg[k])
            v[0] = jnp.where(c[0], ds, v[0])
            sg[0] = jnp.where(c[0], fs, sg[0])
        for k in range(_K):
            stv_ref[rows, k * _L:(k + 1) * _L] = v[k]
            sts_ref[rows, k * _L:(k + 1) * _L] = sg[k]
        return carry

    lax.fori_loop(0, _NSLAB, slab_body, 0)

    @pl.when(i == _NCHUNK - 1)
    def _emit():
        vals = stv_ref[...]
        lane = (lax.broadcasted_iota(jnp.int32, (_B, _K * _L), 1)
                & (_L - 1)).astype(jnp.float32)
        gids = sts_ref[...] * float(_L) + lane
        bv, bi = _extract5(vals, gids)
        eye = (lax.broadcasted_iota(jnp.int32, (_B, _B), 0) ==
               lax.broadcasted_iota(jnp.int32, (_B, _B), 1)).astype(jnp.float32)
        bi_t = lax.dot_general(bi, eye, (((0,), (0,)), ((), ())),
                               preferred_element_type=jnp.float32)
        rows = lax.broadcasted_iota(jnp.int32, (8, _B), 0)
        bi_t = jnp.where(rows < _K, bi_t, 0.0)
        out_ref[...] = bi_t[:_IDXROWS, :].astype(jnp.int32)


def _knn_top5(t_f, gallery):
    return pl.pallas_call(
        _topk_body,
        grid=(_NCHUNK,),
        in_specs=[
            pl.BlockSpec((_B, _L), lambda i: (0, 0)),
            pl.BlockSpec((_CHUNK, _L), lambda i: (i, 0)),
        ],
        out_specs=pl.BlockSpec((_IDXROWS, _B), lambda i: (0, 0)),
        out_shape=jax.ShapeDtypeStruct((_IDXROWS, _B), jnp.int32),
        scratch_shapes=[
            pltpu.VMEM((_B, _K * _L), jnp.float32),
            pltpu.VMEM((_B, _K * _L), jnp.float32),
            pltpu.VMEM((_B, _CHUNK), jnp.float32),
            pltpu.VMEM((_B, 1), jnp.float32),
        ],
    )(t_f, gallery)


def _gather_rows(gallery, idx):
    mesh = plsc.VectorSubcoreMesh(core_axis_name="c", subcore_axis_name="s")

    @functools.partial(
        pl.kernel,
        mesh=mesh,
        out_type=jax.ShapeDtypeStruct((_GROWS, _L), jnp.float32),
        scratch_types=[
            pltpu.VMEM((_BPW,), jnp.int32),
            pltpu.VMEM((_BPW, _L), jnp.float32),
            pltpu.SemaphoreType.DMA,
        ],
    )
    def gather_k(gallery_hbm, idx_hbm, out_hbm, idx_v, rows_v, sem):
        wid = lax.axis_index("s") * 2 + lax.axis_index("c")
        base = wid * _BPW
        pltpu.sync_copy(idx_hbm.at[pl.ds(base, _BPW)], idx_v)
        pltpu.async_copy(gallery_hbm.at[idx_v], rows_v, sem).wait()
        pltpu.sync_copy(rows_v, out_hbm.at[pl.ds(base, _BPW)])

    return gather_k(gallery, idx)


def _mask_body(sim_ref, idx_ref, s_ref, t_ref, mask_ref, out_ref):
    s = s_ref[...]
    t = t_ref[...]
    iota_col = lax.broadcasted_iota(jnp.int32, (_L, 1), 0)
    eye = (lax.broadcasted_iota(jnp.int32, (_L, _L), 0) ==
           lax.broadcasted_iota(jnp.int32, (_L, _L), 1)).astype(jnp.float32)
    idx_all = idx_ref[...]

    blocks = []
    for kk in range(_K):
        rows = sim_ref[kk * _B:(kk + 1) * _B, :]
        idxrow = idx_all[kk:kk + 1, :]
        ohT = (iota_col == idxrow).astype(jnp.float32)
        fix = lax.dot_general(ohT, t, (((0,), (0,)), ((), ())),
                              preferred_element_type=jnp.float32)
        matchrow = jnp.sum(ohT, axis=0, keepdims=True)
        matchcol = lax.dot_general(eye, matchrow, (((1,), (1,)), ((), ())),
                                   preferred_element_type=jnp.float32)
        fixed = fix + rows * (1.0 - matchcol)
        blocks.append(jnp.abs(fixed * s))
    dall = jnp.concatenate(blocks, axis=0)
    lane = lax.broadcasted_iota(jnp.int32, (_K * _B, _L), 1)
    lrow = lax.broadcasted_iota(jnp.int32, (1, _L), 1)
    iota8 = lax.broadcasted_iota(jnp.int32, (1, 8), 1)

    def body(tt, member):
        e8 = (iota_col == tt * 8 + iota8).astype(jnp.float32)
        cols = lax.dot_general(dall, e8, (((1,), (0,)), ((), ())),
                               preferred_element_type=jnp.float32)
        for c in range(8):
            l = tt * 8 + c
            col = cols[:, c:c + 1]
            lt = jnp.sum((dall < col).astype(jnp.float32), axis=1,
                         keepdims=True)
            eqb = jnp.sum(((dall == col) & (lane < l)).astype(jnp.float32),
                          axis=1, keepdims=True)
            intop = ((lt + eqb) < _KEEP).astype(jnp.float32)
            allb = jnp.min(intop)
            member = member * jnp.where(lrow == l, allb, 1.0)
        return member

    member = lax.fori_loop(0, _L // 8, body, jnp.ones((1, _L), jnp.float32))
    maskrow = jnp.where(member > 0.5, 0.0, mask_ref[...])
    maskcol = lax.dot_general(eye, maskrow, (((1,), (1,)), ((), ())),
                              preferred_element_type=jnp.float32)
    out_ref[...] = jnp.broadcast_to(maskcol, (_L, _L))


def _mask_compute(sim, idx2d, s_f, t_f, mask_row):
    return pl.pallas_call(
        _mask_body,
        out_shape=jax.ShapeDtypeStruct((_L, _L), jnp.float32),
    )(sim, idx2d, s_f, t_f, mask_row)


def kernel(s_f, t_f, gallery, mask_init):
    idx2d = _knn_top5(t_f, gallery)
    sim = _gather_rows(gallery, idx2d.reshape(_GROWS))
    out2d = _mask_compute(sim, idx2d, s_f, t_f, mask_init.reshape(1, _L))
    return out2d.reshape(_L, _L, 1, 1)


# speedup vs baseline: 55.7406x; 1.0070x over previous
import functools

import jax
import jax.numpy as jnp
from jax import lax
from jax.experimental import pallas as pl
from jax.experimental.pallas import tpu as pltpu
from jax.experimental.pallas import tpu_sc as plsc

_SIZE = 100000
_B = 128
_L = 128
_K = 5
_KEEP = 64

_CHUNK = 16384
_NCHUNK = (_SIZE + _CHUNK - 1) // _CHUNK
_NSEG = _CHUNK // _L

_INF = float("inf")
_BIGID = float(2**30)

_IDXROWS = 6
_GROWS = _IDXROWS * _B
_NW = 32
_BPW = _GROWS // _NW


def _extract5(vals, ids):
    cols_v, cols_i = [], []
    for _ in range(_K):
        m = jnp.min(vals, axis=1, keepdims=True)
        pick = jnp.min(jnp.where(vals == m, ids, _BIGID), axis=1, keepdims=True)
        cols_v.append(m)
        cols_i.append(pick)
        vals = jnp.where(ids == pick, _INF, vals)
    pad_v = jnp.full((_B, 8 - _K), _INF, jnp.float32)
    pad_i = jnp.full((_B, 8 - _K), _BIGID, jnp.float32)
    return (jnp.concatenate(cols_v + [pad_v], axis=1),
            jnp.concatenate(cols_i + [pad_i], axis=1))


_SLAB = 8
_NSLAB = _B // _SLAB


def _topk_body(t_ref, g_ref, out_ref, stv_ref, sts_ref, mm_ref, xx_ref):
    i = pl.program_id(0)
    t = t_ref[...]
    xx = jnp.sum(t * t, axis=1, keepdims=True)
    ones_row = jnp.ones((1, _L), jnp.float32)

    @pl.when(i == 0)
    def _init():
        mm0 = lax.dot_general(t, t, (((1,), (1,)), ((), ())),
                              preferred_element_type=jnp.float32)
        xs_row = lax.dot_general(ones_row, t * t, (((1,), (1,)), ((), ())),
                                 preferred_element_type=jnp.float32)
        d0 = jnp.maximum(xx + xs_row - 2.0 * mm0, 1e-12)
        stv_ref[:, 0:_L] = d0
        stv_ref[:, _L:] = jnp.full((_B, 4 * _L), _INF, jnp.float32)
        sts_ref[...] = jnp.zeros((_B, _K * _L), jnp.float32)

    g = g_ref[...]
    mm_ref[...] = lax.dot_general(t, g, (((1,), (1,)), ((), ())),
                                  preferred_element_type=jnp.float32)
    yy_row = lax.dot_general(ones_row, g * g, (((1,), (1,)), ((), ())),
                             preferred_element_type=jnp.float32)
    xx_ref[...] = xx
    base = (i * _CHUNK).astype(jnp.float32)
    segbase = (i * _NSEG).astype(jnp.float32)
    iota8 = lax.broadcasted_iota(jnp.int32, (_SLAB, _L), 1).astype(jnp.float32)

    def slab_body(sl, carry):
        rows = pl.ds(sl * _SLAB, _SLAB)
        xx8 = xx_ref[rows, :]
        v = [stv_ref[rows, k * _L:(k + 1) * _L] for k in range(_K)]
        sg = [sts_ref[rows, k * _L:(k + 1) * _L] for k in range(_K)]
        for s in range(_NSEG):
            ids_s = base + (float(s * _L) + iota8)
            mmsl = mm_ref[rows, s * _L:(s + 1) * _L]
            ds = xx8 + yy_row[:, s * _L:(s + 1) * _L] - 2.0 * mmsl
            ds = jnp.maximum(ds, 1e-12)
            ds = jnp.where((ids_s >= float(_B)) & (ids_s < float(_SIZE)),
                           ds, _INF)
            fs = segbase + float(s)
            c = [ds < v[k] for k in range(_K)]
            for k in range(_K - 1, 0, -1):
                v[k] = jnp.where(c[k], jnp.where(c[k - 1], v[k - 1], ds), v[k])
                sg[k] = jnp.where(c[k], jnp.where(c[k - 1], sg[k - 1], fs),
                                  sg[k])
            v[0] = jnp.where(c[0], ds, v[0])
            sg[0] = jnp.where(c[0], fs, sg[0])
        for k in range(_K):
            stv_ref[rows, k * _L:(k + 1) * _L] = v[k]
            sts_ref[rows, k * _L:(k + 1) * _L] = sg[k]
        return carry

    lax.fori_loop(0, _NSLAB, slab_body, 0)

    @pl.when(i == _NCHUNK - 1)
    def _emit():
        vals = stv_ref[...]
        lane = (lax.broadcasted_iota(jnp.int32, (_B, _K * _L), 1)
                & (_L - 1)).astype(jnp.float32)
        gids = sts_ref[...] * float(_L) + lane
        bv, bi = _extract5(vals, gids)
        eye = (lax.broadcasted_iota(jnp.int32, (_B, _B), 0) ==
               lax.broadcasted_iota(jnp.int32, (_B, _B), 1)).astype(jnp.float32)
        bi_t = lax.dot_general(bi, eye, (((0,), (0,)), ((), ())),
                               preferred_element_type=jnp.float32)
        rows = lax.broadcasted_iota(jnp.int32, (8, _B), 0)
        bi_t = jnp.where(rows < _K, bi_t, 0.0)
        out_ref[...] = bi_t[:_IDXROWS, :].astype(jnp.int32)


def _knn_top5(t_f, gallery):
    return pl.pallas_call(
        _topk_body,
        grid=(_NCHUNK,),
        in_specs=[
            pl.BlockSpec((_B, _L), lambda i: (0, 0)),
            pl.BlockSpec((_CHUNK, _L), lambda i: (i, 0)),
        ],
        out_specs=pl.BlockSpec((_IDXROWS, _B), lambda i: (0, 0)),
        out_shape=jax.ShapeDtypeStruct((_IDXROWS, _B), jnp.int32),
        scratch_shapes=[
            pltpu.VMEM((_B, _K * _L), jnp.float32),
            pltpu.VMEM((_B, _K * _L), jnp.float32),
            pltpu.VMEM((_B, _CHUNK), jnp.float32),
            pltpu.VMEM((_B, 1), jnp.float32),
        ],
    )(t_f, gallery)


def _gather_rows(gallery, idx):
    mesh = plsc.VectorSubcoreMesh(core_axis_name="c", subcore_axis_name="s")

    @functools.partial(
        pl.kernel,
        mesh=mesh,
        out_type=jax.ShapeDtypeStruct((_GROWS, _L), jnp.float32),
        scratch_types=[
            pltpu.VMEM((_BPW,), jnp.int32),
            pltpu.VMEM((_BPW, _L), jnp.float32),
            pltpu.SemaphoreType.DMA,
        ],
    )
    def gather_k(gallery_hbm, idx_hbm, out_hbm, idx_v, rows_v, sem):
        wid = lax.axis_index("s") * 2 + lax.axis_index("c")
        base = wid * _BPW
        pltpu.sync_copy(idx_hbm.at[pl.ds(base, _BPW)], idx_v)
        pltpu.async_copy(gallery_hbm.at[idx_v], rows_v, sem).wait()
        pltpu.sync_copy(rows_v, out_hbm.at[pl.ds(base, _BPW)])

    return gather_k(gallery, idx)


def _mask_body(sim_ref, idx_ref, s_ref, t_ref, mask_ref, out_ref):
    s = s_ref[...]
    t = t_ref[...]
    iota_col = lax.broadcasted_iota(jnp.int32, (_L, 1), 0)
    eye = (lax.broadcasted_iota(jnp.int32, (_L, _L), 0) ==
           lax.broadcasted_iota(jnp.int32, (_L, _L), 1)).astype(jnp.float32)
    idx_all = idx_ref[...]

    blocks = []
    for kk in range(_K):
        rows = sim_ref[kk * _B:(kk + 1) * _B, :]
        idxrow = idx_all[kk:kk + 1, :]
        ohT = (iota_col == idxrow).astype(jnp.float32)
        fix = lax.dot_general(ohT, t, (((0,), (0,)), ((), ())),
                              preferred_element_type=jnp.float32)
        matchrow = jnp.sum(ohT, axis=0, keepdims=True)
        matchcol = lax.dot_general(eye, matchrow, (((1,), (1,)), ((), ())),
                                   preferred_element_type=jnp.float32)
        fixed = fix + rows * (1.0 - matchcol)
        blocks.append(jnp.abs(fixed * s))
    dall = jnp.concatenate(blocks, axis=0)
    lane = lax.broadcasted_iota(jnp.int32, (_K * _B, _L), 1)
    lrow = lax.broadcasted_iota(jnp.int32, (1, _L), 1)
    iota8 = lax.broadcasted_iota(jnp.int32, (1, 8), 1)

    def body(tt, member):
        e8 = (iota_col == tt * 8 + iota8).astype(jnp.float32)
        cols = lax.dot_general(dall, e8, (((1,), (0,)), ((), ())),
                               preferred_element_type=jnp.float32)
        for c in range(8):
            l = tt * 8 + c
            col = cols[:, c:c + 1]
            lt = jnp.sum((dall < col).astype(jnp.float32), axis=1,
                         keepdims=True)
            eqb = jnp.sum(((dall == col) & (lane < l)).astype(jnp.float32),
                          axis=1, keepdims=True)
            intop = ((lt + eqb) < _KEEP).astype(jnp.float32)
            allb = jnp.min(intop)
            member = member * jnp.where(lrow == l, allb, 1.0)
        return member

    member = lax.fori_loop(0, _L // 8, body, jnp.ones((1, _L), jnp.float32))
    maskrow = jnp.where(member > 0.5, 0.0, mask_ref[...])
    maskcol = lax.dot_general(eye, maskrow, (((1,), (1,)), ((), ())),
                              preferred_element_type=jnp.float32)
    out_ref[...] = jnp.broadcast_to(maskcol, (_L, _L))


def _mask_compute(sim, idx2d, s_f, t_f, mask_row):
    return pl.pallas_call(
        _mask_body,
        out_shape=jax.ShapeDtypeStruct((_L, _L), jnp.float32),
    )(sim, idx2d, s_f, t_f, mask_row)


def kernel(s_f, t_f, gallery, mask_init):
    idx2d = _knn_top5(t_f, gallery)
    sim = _gather_rows(gallery, idx2d.reshape(_GROWS))
    out2d = _mask_compute(sim, idx2d, s_f, t_f, mask_init.reshape(1, _L))
    return out2d.reshape(_L, _L, 1, 1)
